# baseline (speedup 1.0000x reference)
import functools

import jax
import jax.numpy as jnp
from jax.experimental import pallas as pl
from jax.experimental.pallas import tpu as pltpu

F32 = jnp.float32


def _fold_bn(w, b, g, beta):
    s = g / jnp.sqrt(1.0 + 1e-5)
    wf = w * s[:, None, None, None]
    bf = b * s + beta
    return wf, bf


def _conv1_body(xv_hbm, w_ref, b_ref, s_ref, o_ref, sa, sb, sems, *, R, RP):
    b = pl.program_id(0)
    t = pl.program_id(1)
    T = pl.num_programs(1)
    NG = pl.num_programs(0) * T
    g = b * T + t
    slot = jax.lax.rem(g, 2)

    def dma_for(gi, slot_i, ci, par):
        bi = gi // T
        ti = gi - bi * T
        dst = sa if par == 0 else sb
        return pltpu.make_async_copy(
            xv_hbm.at[bi, ci, pl.ds(ti * R, RP), par, :],
            dst.at[slot_i, ci], sems.at[slot_i, ci, par])

    @pl.when(g == 0)
    def _():
        for ci in range(3):
            for par in range(2):
                dma_for(g, slot, ci, par).start()

    for ci in range(3):
        for par in range(2):
            dma_for(g, slot, ci, par).wait()

    @pl.when(g + 1 < NG)
    def _():
        for ci in range(3):
            for par in range(2):
                dma_for(g + 1, 1 - slot, ci, par).start()
    Va = sa[slot].reshape(3 * RP, 512)
    Vb = sb[slot].reshape(3 * RP, 512)
    Sc = s_ref[...]
    A3 = jnp.dot(Va, Sc, preferred_element_type=F32)
    B3 = jnp.dot(Vb, Sc, preferred_element_type=F32)
    wm = w_ref[...]
    bias = b_ref[...]
    for r in range(R):
        pieces = []
        for dy in range(3):
            src, rr = (A3, r) if dy == 0 else ((B3, r) if dy == 1 else (A3, r + 1))
            for dx in range(3):
                for ci in range(3):
                    pieces.append(
                        src[ci * RP + rr:ci * RP + rr + 1, dx * 256:(dx + 1) * 256])
        P = jnp.concatenate(pieces, axis=0)
        res = jax.lax.dot_general(P, wm, (((0,), (0,)), ((), ())),
                                  preferred_element_type=F32)
        o_ref[0, r] = jax.nn.gelu(res + bias)


def _conv_s2_body(x_hbm, w_ref, b_ref, o_ref, scratch, sem, *, R, OW, W2, Ci,
                  Co, act):
    b = pl.program_id(0)
    t = pl.program_id(1)
    T = pl.num_programs(1)
    NG = pl.num_programs(0) * T
    g = b * T + t
    slot = jax.lax.rem(g, 2)

    def dma_for(gi, slot_i, pr, pc):
        bi = gi // T
        ti = gi - bi * T
        return pltpu.make_async_copy(
            x_hbm.at[bi, pl.ds(ti * R, R + 8), pr, :, pc, :],
            scratch.at[slot_i, pr, pc], sem.at[slot_i, pr, pc])

    @pl.when(g == 0)
    def _():
        for pr in range(2):
            for pc in range(2):
                dma_for(g, slot, pr, pc).start()

    for pr in range(2):
        for pc in range(2):
            dma_for(g, slot, pr, pc).wait()

    @pl.when(g + 1 < NG)
    def _():
        for pr in range(2):
            for pc in range(2):
                dma_for(g + 1, 1 - slot, pr, pc).start()

    acc = b_ref[...].astype(F32) * jnp.ones((R * OW, 1), F32)
    for dy in range(3):
        pr, a = dy % 2, dy // 2
        for dx in range(3):
            pc, c0 = dx % 2, dx // 2
            patch = scratch[slot, pr, pc, a:a + R, c0:c0 + OW, :]
            patch = patch.reshape(R * OW, Ci)
            acc = acc + jnp.dot(patch, w_ref[dy, dx], preferred_element_type=F32)
    if act:
        acc = jax.nn.gelu(acc)
    o_ref[...] = acc.reshape(o_ref.shape)


def _conv_s1_body(x_ref, w_ref, b_ref, pos_ref, o_ref):
    acc = b_ref[...].astype(F32) * jnp.ones((1024, 1), F32)
    for dy in range(3):
        for dx in range(3):
            patch = x_ref[0, dy:dy + 32, dx:dx + 32, :].reshape(1024, 192)
            acc = acc + jnp.dot(patch, w_ref[dy, dx], preferred_element_type=F32)
    o_ref[0] = acc + pos_ref[...]


N = 1024
C = 192


def _graph_block(x, k, W1, b1, Wnx, Wnd, bn_, W2, b2, Wf1, bf1, Wf2, bf2):
    y = jnp.dot(x, W1, preferred_element_type=F32) + b1
    x2 = jnp.sum(y * y, axis=1, keepdims=True)
    x2c = jax.lax.dot_general(jnp.ones((1, C), F32), y * y,
                              (((1,), (1,)), ((), ())),
                              preferred_element_type=F32)
    G = jax.lax.dot_general(y, y, (((1,), (1,)), ((), ())),
                            preferred_element_type=F32)
    dist = x2 + x2c - 2.0 * G
    col = jax.lax.broadcasted_iota(jnp.int32, (N, N), 1)

    y_hi = y.astype(jnp.bfloat16)
    y_lo = (y - y_hi.astype(F32)).astype(jnp.bfloat16)

    def body(t, carry):
        work, maxf = carry
        m = jnp.min(work, axis=1, keepdims=True)
        idxm = jnp.where(work == m, col, jnp.int32(2 ** 30))
        amin = jnp.min(idxm, axis=1, keepdims=True)
        onehot = (col == amin)
        ohb = onehot.astype(jnp.bfloat16)
        sel = (jnp.dot(ohb, y_hi, preferred_element_type=F32)
               + jnp.dot(ohb, y_lo, preferred_element_type=F32))
        maxf = jnp.maximum(maxf, sel)
        work = jnp.where(onehot, F32(1e30), work)
        return work, maxf

    _, maxf = jax.lax.fori_loop(
        0, k, body, (dist, jnp.full((N, C), -1e30, F32)))
    diff = maxf - y
    h = (jnp.dot(y, Wnx, preferred_element_type=F32)
         + jnp.dot(diff, Wnd, preferred_element_type=F32) + bn_)
    h = jax.nn.gelu(h)
    out = jnp.dot(h, W2, preferred_element_type=F32) + b2 + x
    t = jax.nn.gelu(jnp.dot(out, Wf1, preferred_element_type=F32) + bf1)
    return jnp.dot(t, Wf2, preferred_element_type=F32) + bf2 + out


def _net_body(x_ref,
              aW1, ab1, aWnx, aWnd, abn, aW2, ab2, aWf1, abf1, aWf2, abf2,
              bW1, bb1, bWnx, bWnd, bbn, bW2, bb2, bWf1, bbf1, bWf2, bbf2,
              Wp, bp, Wh, bh, o_ref):
    x = x_ref[0]
    x = _graph_block(x, 9, aW1[...], ab1[...], aWnx[...], aWnd[...], abn[...],
                     aW2[...], ab2[...], aWf1[...], abf1[...], aWf2[...], abf2[...])
    x = _graph_block(x, 18, bW1[...], bb1[...], bWnx[...], bWnd[...], bbn[...],
                     bW2[...], bb2[...], bWf1[...], bbf1[...], bWf2[...], bbf2[...])
    m = jax.lax.dot_general(jnp.ones((1, N), F32), x,
                            (((1,), (0,)), ((), ())),
                            preferred_element_type=F32) * F32(1.0 / N)
    p = jax.nn.gelu(jnp.dot(m, Wp[...], preferred_element_type=F32) + bp[...])
    o_ref[0] = jnp.dot(p, Wh[...], preferred_element_type=F32) + bh[...]


def _full_spec(shape):
    nd = len(shape)
    return pl.BlockSpec(shape, lambda b: (0,) * nd)


def _full_spec2(shape):
    nd = len(shape)
    return pl.BlockSpec(shape, lambda b, t: (0,) * nd)


def _mm_1x1(w):
    return w[:, :, 0, 0].T


def kernel(x, params):
    B = x.shape[0]

    st = params['stem']
    wf, bf = zip(*[_fold_bn(l['w'], l['b'], l['g'], l['beta']) for l in st])

    def conv_s2(xin, w, b, OH, Ci, Co, act, R):
        H2 = OH + 1
        xp = jnp.pad(xin, ((0, 0), (1, 15), (1, 1), (0, 0)))
        xv4 = xp.reshape(B, OH + 8, 2, H2, 2, Ci)
        wt = w.transpose(2, 3, 1, 0)
        bt = b.reshape(1, Co)
        T = OH // R
        body = functools.partial(_conv_s2_body, R=R, OW=OH, W2=H2,
                                 Ci=Ci, Co=Co, act=act)
        return pl.pallas_call(
            body,
            grid=(B, T),
            in_specs=[pl.BlockSpec(memory_space=pl.ANY),
                      _full_spec2(wt.shape), _full_spec2(bt.shape)],
            out_specs=pl.BlockSpec((1, R, OH, Co), lambda b, t: (b, t, 0, 0)),
            out_shape=jax.ShapeDtypeStruct((B, OH, OH, Co), F32),
            scratch_shapes=[pltpu.VMEM((2, 2, 2, R + 8, H2, Ci), F32),
                            pltpu.SemaphoreType.DMA((2, 2, 2))],
            compiler_params=pltpu.CompilerParams(vmem_limit_bytes=100 * 1024 * 1024),
        )(xv4, wt, bt)

    xv = jnp.pad(x, ((0, 0), (0, 0), (1, 15), (0, 0))).reshape(B, 3, 264, 2, 512)
    w1 = wf[0].transpose(2, 3, 1, 0).reshape(9 * 3, 64)
    b1 = bf[0].reshape(1, 64)
    wi = jnp.arange(512, dtype=jnp.int32)[:, None]
    oxi = jnp.arange(256, dtype=jnp.int32)[None, :]
    s_cat = jnp.concatenate(
        [(wi == 2 * oxi + dx - 1).astype(F32) for dx in range(3)], axis=1)
    R1, RP1 = 32, 40
    h1 = pl.pallas_call(
        functools.partial(_conv1_body, R=R1, RP=RP1),
        grid=(B, 256 // R1),
        in_specs=[pl.BlockSpec(memory_space=pl.ANY),
                  _full_spec2(w1.shape), _full_spec2(b1.shape),
                  _full_spec2(s_cat.shape)],
        out_specs=pl.BlockSpec((1, R1, 256, 64), lambda b, t: (b, t, 0, 0)),
        out_shape=jax.ShapeDtypeStruct((B, 256, 256, 64), F32),
        scratch_shapes=[pltpu.VMEM((2, 3, RP1, 512), F32),
                        pltpu.VMEM((2, 3, RP1, 512), F32),
                        pltpu.SemaphoreType.DMA((2, 3, 2))],
        compiler_params=pltpu.CompilerParams(vmem_limit_bytes=100 * 1024 * 1024),
    )(xv, w1, b1, s_cat)
    h2 = conv_s2(h1, wf[1], bf[1], 128, 64, 64, True, R=32)
    h3 = conv_s2(h2, wf[2], bf[2], 64, 64, 96, True, R=64)
    h4 = conv_s2(h3, wf[3], bf[3], 32, 96, 192, True, R=32)

    xp5 = jnp.pad(h4, ((0, 0), (1, 1), (1, 1), (0, 0)))
    w5 = wf[4].transpose(2, 3, 1, 0)
    b5 = bf[4].reshape(1, 192)
    pos = params['pos'][0].reshape(192, 1024).T
    nodes = pl.pallas_call(
        _conv_s1_body,
        grid=(B,),
        in_specs=[pl.BlockSpec((1, 34, 34, 192), lambda b: (b, 0, 0, 0)),
                  _full_spec(w5.shape), _full_spec(b5.shape),
                  _full_spec(pos.shape)],
        out_specs=pl.BlockSpec((1, 1024, 192), lambda b: (b, 0, 0)),
        out_shape=jax.ShapeDtypeStruct((B, 1024, 192), F32),
        compiler_params=pltpu.CompilerParams(vmem_limit_bytes=120 * 1024 * 1024),
    )(xp5, w5, b5, pos)

    def block_args(blk):
        W1, b1_ = _fold_bn(blk['g_fc1_w'], blk['g_fc1_b'], blk['g_fc1_g'], blk['g_fc1_beta'])
        Wn, bn_ = _fold_bn(blk['g_nn_w'], blk['g_nn_b'], blk['g_nn_g'], blk['g_nn_beta'])
        W2, b2_ = _fold_bn(blk['g_fc2_w'], blk['g_fc2_b'], blk['g_fc2_g'], blk['g_fc2_beta'])
        Wf1, bf1_ = _fold_bn(blk['f_fc1_w'], blk['f_fc1_b'], blk['f_fc1_g'], blk['f_fc1_beta'])
        Wf2, bf2_ = _fold_bn(blk['f_fc2_w'], blk['f_fc2_b'], blk['f_fc2_g'], blk['f_fc2_beta'])
        Wnm = _mm_1x1(Wn)
        return (_mm_1x1(W1), b1_.reshape(1, C),
                Wnm[:C], Wnm[C:], bn_.reshape(1, 2 * C),
                _mm_1x1(W2), b2_.reshape(1, C),
                _mm_1x1(Wf1), bf1_.reshape(1, 4 * C),
                _mm_1x1(Wf2), bf2_.reshape(1, C))

    argsA = block_args(params['blocks'][0])
    argsB = block_args(params['blocks'][1])
    Wpm, bpm = _fold_bn(params['pred_w'], params['pred_b'], params['pred_g'], params['pred_beta'])
    head_args = (_mm_1x1(Wpm), bpm.reshape(1, 1024),
                 _mm_1x1(params['head_w']), params['head_b'].reshape(1, 1000))

    wargs = argsA + argsB + head_args
    logits = pl.pallas_call(
        _net_body,
        grid=(B,),
        in_specs=[pl.BlockSpec((1, 1024, 192), lambda b: (b, 0, 0))] +
                 [_full_spec(a.shape) for a in wargs],
        out_specs=pl.BlockSpec((1, 1, 1000), lambda b: (b, 0, 0)),
        out_shape=jax.ShapeDtypeStruct((B, 1, 1000), F32),
        compiler_params=pltpu.CompilerParams(vmem_limit_bytes=120 * 1024 * 1024),
    )(nodes, *wargs)
    return logits.reshape(B, 1000)


# speedup vs baseline: 1.0643x; 1.0643x over previous
import functools

import jax
import jax.numpy as jnp
from jax.experimental import pallas as pl
from jax.experimental.pallas import tpu as pltpu

F32 = jnp.float32


def _fold_bn(w, b, g, beta):
    s = g / jnp.sqrt(1.0 + 1e-5)
    wf = w * s[:, None, None, None]
    bf = b * s + beta
    return wf, bf


def _conv1_body(xv_hbm, w_ref, b_ref, s_ref, o_ref, sa, sb, sems, *, R, RP):
    b = pl.program_id(0)
    t = pl.program_id(1)
    T = pl.num_programs(1)
    NG = pl.num_programs(0) * T
    g = b * T + t
    slot = jax.lax.rem(g, 2)

    def dma_for(gi, slot_i, ci, par):
        bi = gi // T
        ti = gi - bi * T
        dst = sa if par == 0 else sb
        return pltpu.make_async_copy(
            xv_hbm.at[bi, ci, pl.ds(ti * R, RP), par, :],
            dst.at[slot_i, ci], sems.at[slot_i, ci, par])

    @pl.when(g == 0)
    def _():
        for ci in range(3):
            for par in range(2):
                dma_for(g, slot, ci, par).start()

    for ci in range(3):
        for par in range(2):
            dma_for(g, slot, ci, par).wait()

    @pl.when(g + 1 < NG)
    def _():
        for ci in range(3):
            for par in range(2):
                dma_for(g + 1, 1 - slot, ci, par).start()
    Va = sa[slot].reshape(3 * RP, 512)
    Vb = sb[slot].reshape(3 * RP, 512)
    Sc = s_ref[...]
    A3 = jnp.dot(Va, Sc, preferred_element_type=F32)
    B3 = jnp.dot(Vb, Sc, preferred_element_type=F32)
    wm = w_ref[...]
    bias = b_ref[...]
    for r in range(R):
        pieces = []
        for dy in range(3):
            src, rr = (A3, r) if dy == 0 else ((B3, r) if dy == 1 else (A3, r + 1))
            for dx in range(3):
                for ci in range(3):
                    pieces.append(
                        src[ci * RP + rr:ci * RP + rr + 1, dx * 256:(dx + 1) * 256])
        P = jnp.concatenate(pieces, axis=0)
        res = jax.lax.dot_general(P, wm, (((0,), (0,)), ((), ())),
                                  preferred_element_type=F32)
        o_ref[0, r] = jax.nn.gelu(res + bias)


def _conv_s2_body(x_hbm, w_ref, b_ref, o_ref, scratch, sem, *, R, OW, W2, Ci,
                  Co, act):
    b = pl.program_id(0)
    t = pl.program_id(1)
    T = pl.num_programs(1)
    NG = pl.num_programs(0) * T
    g = b * T + t
    slot = jax.lax.rem(g, 2)

    def dma_for(gi, slot_i, pr, pc):
        bi = gi // T
        ti = gi - bi * T
        return pltpu.make_async_copy(
            x_hbm.at[bi, pl.ds(ti * R, R + 8), pr, :, pc, :],
            scratch.at[slot_i, pr, pc], sem.at[slot_i, pr, pc])

    @pl.when(g == 0)
    def _():
        for pr in range(2):
            for pc in range(2):
                dma_for(g, slot, pr, pc).start()

    for pr in range(2):
        for pc in range(2):
            dma_for(g, slot, pr, pc).wait()

    @pl.when(g + 1 < NG)
    def _():
        for pr in range(2):
            for pc in range(2):
                dma_for(g + 1, 1 - slot, pr, pc).start()

    acc = b_ref[...].astype(F32) * jnp.ones((R * OW, 1), F32)
    for dy in range(3):
        pr, a = dy % 2, dy // 2
        for dx in range(3):
            pc, c0 = dx % 2, dx // 2
            patch = scratch[slot, pr, pc, a:a + R, c0:c0 + OW, :]
            patch = patch.reshape(R * OW, Ci)
            acc = acc + jnp.dot(patch, w_ref[dy, dx], preferred_element_type=F32)
    if act:
        acc = jax.nn.gelu(acc)
    o_ref[...] = acc.reshape(o_ref.shape)


def _conv_s1_body(x_ref, w_ref, b_ref, pos_ref, o_ref):
    acc = b_ref[...].astype(F32) * jnp.ones((1024, 1), F32)
    for dy in range(3):
        for dx in range(3):
            patch = x_ref[0, dy:dy + 32, dx:dx + 32, :].reshape(1024, 192)
            acc = acc + jnp.dot(patch, w_ref[dy, dx], preferred_element_type=F32)
    o_ref[0] = acc + pos_ref[...]


N = 1024
C = 192


def _graph_block(x, k, W1, b1, Wnx, Wnd, bn_, W2, b2, Wf1, bf1, Wf2, bf2):
    y = jnp.dot(x, W1, preferred_element_type=F32) + b1
    x2 = jnp.sum(y * y, axis=1, keepdims=True)
    x2c = jax.lax.dot_general(jnp.ones((1, C), F32), y * y,
                              (((1,), (1,)), ((), ())),
                              preferred_element_type=F32)
    G = jax.lax.dot_general(y, y, (((1,), (1,)), ((), ())),
                            preferred_element_type=F32)
    dist = x2 + x2c - 2.0 * G
    col = jax.lax.broadcasted_iota(jnp.int32, (N, N), 1)

    def body(t, carry):
        work, maxf = carry
        m = jnp.min(work, axis=1, keepdims=True)
        idxm = jnp.where(work == m, col, jnp.int32(2 ** 30))
        amin = jnp.min(idxm, axis=1, keepdims=True)
        onehot = (col == amin)
        sel = jnp.dot(onehot.astype(F32), y, preferred_element_type=F32)
        maxf = jnp.maximum(maxf, sel)
        work = jnp.where(onehot, F32(1e30), work)
        return work, maxf

    _, maxf = jax.lax.fori_loop(
        0, k, body, (dist, jnp.full((N, C), -1e30, F32)))
    diff = maxf - y
    h = (jnp.dot(y, Wnx, preferred_element_type=F32)
         + jnp.dot(diff, Wnd, preferred_element_type=F32) + bn_)
    h = jax.nn.gelu(h)
    out = jnp.dot(h, W2, preferred_element_type=F32) + b2 + x
    t = jax.nn.gelu(jnp.dot(out, Wf1, preferred_element_type=F32) + bf1)
    return jnp.dot(t, Wf2, preferred_element_type=F32) + bf2 + out


def _net_body(x_ref,
              aW1, ab1, aWnx, aWnd, abn, aW2, ab2, aWf1, abf1, aWf2, abf2,
              bW1, bb1, bWnx, bWnd, bbn, bW2, bb2, bWf1, bbf1, bWf2, bbf2,
              Wp, bp, Wh, bh, o_ref):
    x = x_ref[0]
    x = _graph_block(x, 9, aW1[...], ab1[...], aWnx[...], aWnd[...], abn[...],
                     aW2[...], ab2[...], aWf1[...], abf1[...], aWf2[...], abf2[...])
    x = _graph_block(x, 18, bW1[...], bb1[...], bWnx[...], bWnd[...], bbn[...],
                     bW2[...], bb2[...], bWf1[...], bbf1[...], bWf2[...], bbf2[...])
    m = jax.lax.dot_general(jnp.ones((1, N), F32), x,
                            (((1,), (0,)), ((), ())),
                            preferred_element_type=F32) * F32(1.0 / N)
    p = jax.nn.gelu(jnp.dot(m, Wp[...], preferred_element_type=F32) + bp[...])
    o_ref[0] = jnp.dot(p, Wh[...], preferred_element_type=F32) + bh[...]


def _full_spec(shape):
    nd = len(shape)
    return pl.BlockSpec(shape, lambda b: (0,) * nd)


def _full_spec2(shape):
    nd = len(shape)
    return pl.BlockSpec(shape, lambda b, t: (0,) * nd)


def _mm_1x1(w):
    return w[:, :, 0, 0].T


def kernel(x, params):
    B = x.shape[0]

    st = params['stem']
    wf, bf = zip(*[_fold_bn(l['w'], l['b'], l['g'], l['beta']) for l in st])

    def conv_s2(xin, w, b, OH, Ci, Co, act, R):
        H2 = OH + 1
        xp = jnp.pad(xin, ((0, 0), (1, 15), (1, 1), (0, 0)))
        xv4 = xp.reshape(B, OH + 8, 2, H2, 2, Ci)
        wt = w.transpose(2, 3, 1, 0)
        bt = b.reshape(1, Co)
        T = OH // R
        body = functools.partial(_conv_s2_body, R=R, OW=OH, W2=H2,
                                 Ci=Ci, Co=Co, act=act)
        return pl.pallas_call(
            body,
            grid=(B, T),
            in_specs=[pl.BlockSpec(memory_space=pl.ANY),
                      _full_spec2(wt.shape), _full_spec2(bt.shape)],
            out_specs=pl.BlockSpec((1, R, OH, Co), lambda b, t: (b, t, 0, 0)),
            out_shape=jax.ShapeDtypeStruct((B, OH, OH, Co), F32),
            scratch_shapes=[pltpu.VMEM((2, 2, 2, R + 8, H2, Ci), F32),
                            pltpu.SemaphoreType.DMA((2, 2, 2))],
            compiler_params=pltpu.CompilerParams(vmem_limit_bytes=100 * 1024 * 1024),
        )(xv4, wt, bt)

    xv = jnp.pad(x, ((0, 0), (0, 0), (1, 15), (0, 0))).reshape(B, 3, 264, 2, 512)
    w1 = wf[0].transpose(2, 3, 1, 0).reshape(9 * 3, 64)
    b1 = bf[0].reshape(1, 64)
    wi = jnp.arange(512, dtype=jnp.int32)[:, None]
    oxi = jnp.arange(256, dtype=jnp.int32)[None, :]
    s_cat = jnp.concatenate(
        [(wi == 2 * oxi + dx - 1).astype(F32) for dx in range(3)], axis=1)
    R1, RP1 = 64, 72
    h1 = pl.pallas_call(
        functools.partial(_conv1_body, R=R1, RP=RP1),
        grid=(B, 256 // R1),
        in_specs=[pl.BlockSpec(memory_space=pl.ANY),
                  _full_spec2(w1.shape), _full_spec2(b1.shape),
                  _full_spec2(s_cat.shape)],
        out_specs=pl.BlockSpec((1, R1, 256, 64), lambda b, t: (b, t, 0, 0)),
        out_shape=jax.ShapeDtypeStruct((B, 256, 256, 64), F32),
        scratch_shapes=[pltpu.VMEM((2, 3, RP1, 512), F32),
                        pltpu.VMEM((2, 3, RP1, 512), F32),
                        pltpu.SemaphoreType.DMA((2, 3, 2))],
        compiler_params=pltpu.CompilerParams(vmem_limit_bytes=100 * 1024 * 1024),
    )(xv, w1, b1, s_cat)
    h2 = conv_s2(h1, wf[1], bf[1], 128, 64, 64, True, R=32)
    h3 = conv_s2(h2, wf[2], bf[2], 64, 64, 96, True, R=64)
    h4 = conv_s2(h3, wf[3], bf[3], 32, 96, 192, True, R=32)

    xp5 = jnp.pad(h4, ((0, 0), (1, 1), (1, 1), (0, 0)))
    w5 = wf[4].transpose(2, 3, 1, 0)
    b5 = bf[4].reshape(1, 192)
    pos = params['pos'][0].reshape(192, 1024).T
    nodes = pl.pallas_call(
        _conv_s1_body,
        grid=(B,),
        in_specs=[pl.BlockSpec((1, 34, 34, 192), lambda b: (b, 0, 0, 0)),
                  _full_spec(w5.shape), _full_spec(b5.shape),
                  _full_spec(pos.shape)],
        out_specs=pl.BlockSpec((1, 1024, 192), lambda b: (b, 0, 0)),
        out_shape=jax.ShapeDtypeStruct((B, 1024, 192), F32),
        compiler_params=pltpu.CompilerParams(vmem_limit_bytes=120 * 1024 * 1024),
    )(xp5, w5, b5, pos)

    def block_args(blk):
        W1, b1_ = _fold_bn(blk['g_fc1_w'], blk['g_fc1_b'], blk['g_fc1_g'], blk['g_fc1_beta'])
        Wn, bn_ = _fold_bn(blk['g_nn_w'], blk['g_nn_b'], blk['g_nn_g'], blk['g_nn_beta'])
        W2, b2_ = _fold_bn(blk['g_fc2_w'], blk['g_fc2_b'], blk['g_fc2_g'], blk['g_fc2_beta'])
        Wf1, bf1_ = _fold_bn(blk['f_fc1_w'], blk['f_fc1_b'], blk['f_fc1_g'], blk['f_fc1_beta'])
        Wf2, bf2_ = _fold_bn(blk['f_fc2_w'], blk['f_fc2_b'], blk['f_fc2_g'], blk['f_fc2_beta'])
        Wnm = _mm_1x1(Wn)
        return (_mm_1x1(W1), b1_.reshape(1, C),
                Wnm[:C], Wnm[C:], bn_.reshape(1, 2 * C),
                _mm_1x1(W2), b2_.reshape(1, C),
                _mm_1x1(Wf1), bf1_.reshape(1, 4 * C),
                _mm_1x1(Wf2), bf2_.reshape(1, C))

    argsA = block_args(params['blocks'][0])
    argsB = block_args(params['blocks'][1])
    Wpm, bpm = _fold_bn(params['pred_w'], params['pred_b'], params['pred_g'], params['pred_beta'])
    head_args = (_mm_1x1(Wpm), bpm.reshape(1, 1024),
                 _mm_1x1(params['head_w']), params['head_b'].reshape(1, 1000))

    wargs = argsA + argsB + head_args
    logits = pl.pallas_call(
        _net_body,
        grid=(B,),
        in_specs=[pl.BlockSpec((1, 1024, 192), lambda b: (b, 0, 0))] +
                 [_full_spec(a.shape) for a in wargs],
        out_specs=pl.BlockSpec((1, 1, 1000), lambda b: (b, 0, 0)),
        out_shape=jax.ShapeDtypeStruct((B, 1, 1000), F32),
        compiler_params=pltpu.CompilerParams(vmem_limit_bytes=120 * 1024 * 1024),
    )(nodes, *wargs)
    return logits.reshape(B, 1000)
